# Initial kernel scaffold; baseline (speedup 1.0000x reference)
import jax
import jax.numpy as jnp
from jax.experimental import pallas as pl


def kernel(question, word2vec):
    raise NotImplementedError("write your pallas kernel here")


# speedup vs baseline: 1.5689x; 1.5689x over previous
import functools

import jax
import jax.numpy as jnp
from jax import lax
from jax.experimental import pallas as pl
from jax.experimental.pallas import tpu as pltpu
from jax.experimental.pallas import tpu_sc as plsc

B = 16384
S = 50
D = 32
TOTAL = B * S

NC = 2
NS = 16
NW = NC * NS
PER_W = TOTAL // NW

IPG = 128
CHUNK = 1024
GPC = CHUNK // IPG
N_CHUNKS = PER_W // CHUNK
ROWS_PER_IT = 4


def _tanh16(x):
    e = jnp.exp(x + x)
    return 1.0 - 2.0 / (e + 1.0)


@functools.partial(
    pl.kernel,
    out_type=jax.ShapeDtypeStruct((TOTAL, D), jnp.float32),
    mesh=plsc.VectorSubcoreMesh(core_axis_name="c", subcore_axis_name="s"),
    scratch_types=[
        pltpu.VMEM((GPC, IPG), jnp.int32),
        pltpu.VMEM((CHUNK, D), jnp.float32),
        pltpu.SemaphoreType.DMA,
    ],
    compiler_params=pltpu.CompilerParams(use_tc_tiling_on_sc=False),
)
def _gather_tanh(idx_hbm, tab_hbm, out_hbm, idx_v, rows_v, gsem):
    c = lax.axis_index("c")
    s = lax.axis_index("s")
    wid = s * NC + c
    row0_w = wid * (PER_W // IPG)

    def chunk_body(g, carry):
        irow = row0_w + g * GPC
        base = irow * IPG
        pltpu.sync_copy(idx_hbm.at[pl.ds(irow, GPC)], idx_v)
        copies = []
        for j in range(GPC):
            copies.append(
                pltpu.async_copy(
                    tab_hbm.at[idx_v.at[j]],
                    rows_v.at[pl.ds(j * IPG, IPG)],
                    gsem,
                )
            )
        for cp in copies:
            cp.wait()

        def tanh_body(i, carry2):
            r = i * ROWS_PER_IT
            for u in range(ROWS_PER_IT):
                for h in range(D // 16):
                    x = rows_v[r + u, pl.ds(h * 16, 16)]
                    rows_v[r + u, pl.ds(h * 16, 16)] = _tanh16(x)
            return carry2

        lax.fori_loop(0, CHUNK // ROWS_PER_IT, tanh_body, 0)
        pltpu.sync_copy(rows_v, out_hbm.at[pl.ds(base, CHUNK)])
        return carry

    lax.fori_loop(0, N_CHUNKS, chunk_body, 0)


def kernel(question, word2vec):
    idx = jnp.transpose(question).reshape(TOTAL // IPG, IPG).astype(jnp.int32)
    out = _gather_tanh(idx, word2vec)
    return out.reshape(S, B, D)
